# baseline (speedup 1.0000x reference)
import functools

import jax
import jax.numpy as jnp
from jax.experimental import pallas as pl
from jax.experimental.pallas import tpu as pltpu

_RO = 4.0
_POLY = 128
_NADJ = 4
_K = 9
_RES_DILS = (1, 1, 1, 2, 2, 4, 4)
_ALL_DILS = (1,) + _RES_DILS
_CIN = 128
_STATE = 128
_FUSION = 256
_CAT = 8 * _STATE


def _full_spec(arr):
    zeros = (0,) * arr.ndim
    return pl.BlockSpec(arr.shape, lambda i, _z=zeros: _z)


def _roll_rows(a, s, p):
    if s == 0:
        return a
    return jnp.concatenate([a[:, p - s:, :], a[:, :p - s, :]], axis=1)


def _snake_body(*refs, nb, p, do_fuse, ret_state):
    ix = 0
    feat_ref = refs[ix]; ix += 1
    cv_ref = refs[ix]; ix += 1
    if do_fuse:
        ct_ref = refs[ix]; ix += 1
        wa_ref = refs[ix]; ix += 1
        wb_ref = refs[ix]; ix += 1
        fb_ref = refs[ix]; ix += 1
    (hw_ref, rw_ref, cb_ref, fw_ref, fbias_ref, p0g_ref, p0s_ref, p0b_ref,
     p1_ref, p1b_ref, p2_ref, p2b_ref) = refs[ix:ix + 12]
    ix += 12
    off_ref = refs[ix]; ix += 1
    st_ref = None
    if ret_state:
        st_ref = refs[ix]; ix += 1
    states_sc = refs[ix]

    m = nb * p
    feat = feat_ref[...].reshape(m, 64)
    if do_fuse:
        ct = ct_ref[...].reshape(nb, 64)
        ctv = jnp.dot(ct, wb_ref[...], preferred_element_type=jnp.float32)
        ctb = jnp.broadcast_to(ctv[:, None, :], (nb, p, 64)).reshape(m, 64)
        feat = (jnp.dot(feat, wa_ref[...], preferred_element_type=jnp.float32)
                + ctb + fb_ref[...])
    cv = cv_ref[...].reshape(m, 2)
    x = jnp.concatenate(
        [feat, cv, jnp.zeros((m, _CIN - 66), jnp.float32)], axis=1)

    xb = x.astype(jnp.bfloat16)
    for j, d in enumerate(_ALL_DILS):
        xb3 = xb.reshape(nb, p, _CIN)
        taps = jnp.concatenate(
            [_roll_rows(xb3, (-(k - _NADJ) * d) % p, p) for k in range(_K)],
            axis=2).reshape(m, _K * _CIN)
        wmat = hw_ref[...] if j == 0 else rw_ref[j - 1]
        z = jnp.dot(taps, wmat,
                    preferred_element_type=jnp.float32) + cb_ref[j]
        y = jnp.maximum(z, 0.0)
        x = y if j == 0 else y + x
        xb = x.astype(jnp.bfloat16)
        states_sc[:, j * _STATE:(j + 1) * _STATE] = xb
        if ret_state:
            lo = _FUSION + j * _STATE
            x3 = x.reshape(nb, p, _STATE)
            for b in range(nb):
                st_ref[b, lo:lo + _STATE, :] = jnp.transpose(x3[b])

    states = states_sc[...]
    fus = (jnp.dot(states, fw_ref[...], preferred_element_type=jnp.float32)
           + fbias_ref[...])
    g = jnp.max(fus.reshape(nb, p, _FUSION), axis=1)
    gg = jnp.dot(g.astype(jnp.bfloat16), p0g_ref[...],
                 preferred_element_type=jnp.float32)
    ggb = jnp.broadcast_to(gg[:, None, :], (nb, p, _FUSION)).reshape(m, _FUSION)
    h0 = jnp.maximum(
        jnp.dot(states, p0s_ref[...], preferred_element_type=jnp.float32)
        + ggb + p0b_ref[...], 0.0)
    h1 = jnp.maximum(
        jnp.dot(h0.astype(jnp.bfloat16), p1_ref[...],
                preferred_element_type=jnp.float32) + p1b_ref[...], 0.0)
    h2 = (jnp.dot(h1.astype(jnp.bfloat16), p2_ref[...],
                  preferred_element_type=jnp.float32) + p2b_ref[...])
    off_ref[...] = h2.reshape(nb, p, 2)
    if ret_state:
        for b in range(nb):
            st_ref[b, 0:_FUSION, :] = jnp.broadcast_to(
                jnp.transpose(g[b:b + 1, :]), (_FUSION, p))


def _snake_run(sp, feat_t, cvals, ct=None, fuse_w=None, ret_state=False, nb=8):
    n, p, _ = feat_t.shape
    do_fuse = ct is not None
    body = functools.partial(_snake_body, nb=nb, p=p, do_fuse=do_fuse,
                             ret_state=ret_state)
    inputs = [feat_t, cvals]
    in_specs = [pl.BlockSpec((nb, p, 64), lambda i: (i, 0, 0)),
                pl.BlockSpec((nb, p, 2), lambda i: (i, 0, 0))]
    if do_fuse:
        wa_t, wb_t, fb_t = fuse_w
        inputs += [ct, wa_t, wb_t, fb_t]
        in_specs += [pl.BlockSpec((nb, 1, 64), lambda i: (i, 0, 0)),
                     _full_spec(wa_t), _full_spec(wb_t), _full_spec(fb_t)]
    winputs = [sp['head'], sp['res'], sp['cb'], sp['fw'], sp['fb'],
               sp['p0g'], sp['p0s'], sp['p0b'], sp['p1'], sp['p1b'],
               sp['p2'], sp['p2b']]
    inputs += winputs
    in_specs += [_full_spec(a) for a in winputs]

    out_shape = [jax.ShapeDtypeStruct((n, p, 2), jnp.float32)]
    out_specs = [pl.BlockSpec((nb, p, 2), lambda i: (i, 0, 0))]
    if ret_state:
        out_shape.append(
            jax.ShapeDtypeStruct((n, _FUSION + _CAT, p), jnp.float32))
        out_specs.append(
            pl.BlockSpec((nb, _FUSION + _CAT, p), lambda i: (i, 0, 0)))

    outs = pl.pallas_call(
        body,
        out_shape=out_shape,
        grid=(n // nb,),
        in_specs=in_specs,
        out_specs=out_specs,
        scratch_shapes=[
            pltpu.VMEM((nb * p, _CAT), jnp.bfloat16),
        ],
        compiler_params=pltpu.CompilerParams(
            dimension_semantics=("parallel",),
            vmem_limit_bytes=64 * 1024 * 1024),
    )(*inputs)
    if ret_state:
        return outs[0], outs[1]
    return outs[0]


def _prep_snake(hw, rw, cb, fw, fb, p0g, p0s, p0b, p1w, p1b, p2w, p2b):
    head = jnp.zeros((_K, _CIN, _STATE), jnp.bfloat16)
    head = head.at[:, :66, :].set(jnp.swapaxes(hw, 1, 2))
    return {
        'head': head.reshape(_K * _CIN, _STATE),
        'res': jnp.swapaxes(rw, 1, 2),
        'cb': cb.reshape(8, 1, _STATE),
        'fw': jnp.swapaxes(fw, 0, 1),
        'fb': fb.reshape(1, _FUSION),
        'p0g': jnp.swapaxes(p0g, 0, 1),
        'p0s': jnp.swapaxes(p0s, 0, 1),
        'p0b': p0b.reshape(1, _FUSION),
        'p1': jnp.swapaxes(p1w, 0, 1),
        'p1b': p1b.reshape(1, 64),
        'p2': jnp.swapaxes(p2w, 0, 1),
        'p2b': p2b.reshape(1, 2),
    }


def _feat_t(cnn_pairs, img_poly, ind, h, w):
    x = img_poly[..., 0] - 0.5
    y = img_poly[..., 1] - 0.5
    x0 = jnp.floor(x)
    y0 = jnp.floor(y)
    wx1, wx0 = x - x0, 1.0 - (x - x0)
    wy1, wy0 = y - y0, 1.0 - (y - y0)
    vx0 = ((x0 >= 0) & (x0 <= w - 1)).astype(jnp.float32)
    vx1 = ((x0 >= -1) & (x0 <= w - 2)).astype(jnp.float32)
    vy0 = ((y0 >= 0) & (y0 <= h - 1)).astype(jnp.float32)
    vy1 = ((y0 >= -1) & (y0 <= h - 2)).astype(jnp.float32)
    mid = ((x0 >= 0) & (x0 <= w - 2))[..., None]
    x0c = jnp.clip(x0, 0, w - 2).astype(jnp.int32)
    y0c = jnp.clip(y0, 0, h - 1).astype(jnp.int32)
    y1c = jnp.clip(y0 + 1.0, 0, h - 1).astype(jnp.int32)
    base = ind.astype(jnp.int32)[:, None] * (h * w)
    cw0 = (wx0 * vx0)[..., None]
    cw1 = (wx1 * vx1)[..., None]

    def rowmix(yc):
        pair = cnn_pairs[base + yc * w + x0c]
        a, b = pair[..., :64], pair[..., 64:]
        px0 = jnp.where(mid, a, b)
        px1 = jnp.where(mid, b, a)
        return px0 * cw0 + px1 * cw1

    return (rowmix(y0c) * (wy0 * vy0)[..., None]
            + rowmix(y1c) * (wy1 * vy1)[..., None])


def _can_poly(img_poly):
    x_min = jnp.min(img_poly[..., 0], axis=-1, keepdims=True)
    y_min = jnp.min(img_poly[..., 1], axis=-1, keepdims=True)
    return jnp.stack([img_poly[..., 0] - x_min,
                      img_poly[..., 1] - y_min], axis=-1)


def _upsample(poly, p_num):
    n, p, _ = poly.shape
    t = jnp.arange(p_num, dtype=jnp.float32) * (p / p_num)
    i0 = jnp.floor(t).astype(jnp.int32) % p
    i1 = (i0 + 1) % p
    frac = (t - jnp.floor(t))[None, :, None]
    return poly[:, i0] * (1.0 - frac) + poly[:, i1] * frac


def kernel(cnn_feature, i_it_4py, c_it_4py, ind, shift_init, fuse_wa, fuse_wb, fuse_b, g0_head_w, g0_res_w, g0_conv_b, g0_fusion_w, g0_fusion_b, g0_pred0_gw, g0_pred0_sw, g0_pred0_b, g0_pred1_w, g0_pred1_b, g0_pred2_w, g0_pred2_b, g1_head_w, g1_res_w, g1_conv_b, g1_fusion_w, g1_fusion_b, g1_pred0_gw, g1_pred0_sw, g1_pred0_b, g1_pred1_w, g1_pred1_b, g1_pred2_w, g1_pred2_b, g2_head_w, g2_res_w, g2_conv_b, g2_fusion_w, g2_fusion_b, g2_pred0_gw, g2_pred0_sw, g2_pred0_b, g2_pred1_w, g2_pred1_b, g2_pred2_w, g2_pred2_b, ge_head_w, ge_res_w, ge_conv_b, ge_fusion_w, ge_fusion_b, ge_pred0_gw, ge_pred0_sw, ge_pred0_b, ge_pred1_w, ge_pred1_b, ge_pred2_w, ge_pred2_b, ge0_head_w, ge0_res_w, ge0_conv_b, ge0_fusion_w, ge0_fusion_b, ge0_pred0_gw, ge0_pred0_sw, ge0_pred0_b, ge0_pred1_w, ge0_pred1_b, ge0_pred2_w, ge0_pred2_b):
    h, w = cnn_feature.shape[2], cnn_feature.shape[3]
    cnn_rows = jnp.transpose(cnn_feature, (0, 2, 3, 1)).reshape(-1, 64)
    cnn_pairs = jnp.concatenate(
        [cnn_rows, jnp.concatenate([cnn_rows[1:], cnn_rows[:1]], axis=0)],
        axis=1)
    fuse = (jnp.swapaxes(fuse_wa, 0, 1), jnp.swapaxes(fuse_wb, 0, 1),
            fuse_b.reshape(1, 64))
    sps = [
        _prep_snake(g0_head_w, g0_res_w, g0_conv_b, g0_fusion_w, g0_fusion_b,
                    g0_pred0_gw, g0_pred0_sw, g0_pred0_b, g0_pred1_w,
                    g0_pred1_b, g0_pred2_w, g0_pred2_b),
        _prep_snake(g1_head_w, g1_res_w, g1_conv_b, g1_fusion_w, g1_fusion_b,
                    g1_pred0_gw, g1_pred0_sw, g1_pred0_b, g1_pred1_w,
                    g1_pred1_b, g1_pred2_w, g1_pred2_b),
        _prep_snake(g2_head_w, g2_res_w, g2_conv_b, g2_fusion_w, g2_fusion_b,
                    g2_pred0_gw, g2_pred0_sw, g2_pred0_b, g2_pred1_w,
                    g2_pred1_b, g2_pred2_w, g2_pred2_b),
        _prep_snake(ge_head_w, ge_res_w, ge_conv_b, ge_fusion_w, ge_fusion_b,
                    ge_pred0_gw, ge_pred0_sw, ge_pred0_b, ge_pred1_w,
                    ge_pred1_b, ge_pred2_w, ge_pred2_b),
        _prep_snake(ge0_head_w, ge0_res_w, ge0_conv_b, ge0_fusion_w,
                    ge0_fusion_b, ge0_pred0_gw, ge0_pred0_sw, ge0_pred0_b,
                    ge0_pred1_w, ge0_pred1_b, ge0_pred2_w, ge0_pred2_b),
    ]

    ipoly = i_it_4py
    cpoly = c_it_4py
    for it in range(3):
        feat_t = _feat_t(cnn_pairs, ipoly, ind, h, w)
        center = (jnp.min(ipoly, axis=1) + jnp.max(ipoly, axis=1)) * 0.5
        ct_t = _feat_t(cnn_pairs, center[:, None], ind, h, w)
        off = _snake_run(sps[it], feat_t, cpoly, ct=ct_t, fuse_w=fuse, nb=64)
        ipoly = ipoly + off
        cpoly = _can_poly(ipoly)
    ex_pred = ipoly

    i_py = _upsample(ex_pred, _POLY)
    c_py = _can_poly(i_py)
    feat_t = _feat_t(cnn_pairs, i_py, ind, h, w)
    off = _snake_run(sps[3], feat_t, c_py * _RO, nb=32)
    py0 = i_py * _RO + off

    cur = py0 / _RO
    c_cur = _can_poly(cur)
    feat_t = _feat_t(cnn_pairs, cur, ind, h, w)
    off, st_t = _snake_run(sps[4], feat_t, c_cur * _RO, ret_state=True, nb=16)
    py1 = cur * _RO + off
    state = st_t

    return {
        'ex_pred': ex_pred,
        'py_pred': [py0, py1],
        'state': state,
        'py': [py0 / _RO, py1 / _RO],
    }


# speedup vs baseline: 1.0057x; 1.0057x over previous
import functools

import jax
import jax.numpy as jnp
from jax.experimental import pallas as pl
from jax.experimental.pallas import tpu as pltpu

_RO = 4.0
_POLY = 128
_NADJ = 4
_K = 9
_RES_DILS = (1, 1, 1, 2, 2, 4, 4)
_ALL_DILS = (1,) + _RES_DILS
_CIN = 128
_STATE = 128
_FUSION = 256
_CAT = 8 * _STATE


def _full_spec(arr):
    zeros = (0,) * arr.ndim
    return pl.BlockSpec(arr.shape, lambda i, _z=zeros: _z)


def _roll_rows(a, s, p):
    if s == 0:
        return a
    return jnp.concatenate([a[:, p - s:, :], a[:, :p - s, :]], axis=1)


def _snake_body(*refs, nb, p, do_fuse, ret_state):
    ix = 0
    feat_ref = refs[ix]; ix += 1
    cv_ref = refs[ix]; ix += 1
    if do_fuse:
        ct_ref = refs[ix]; ix += 1
        wa_ref = refs[ix]; ix += 1
        wb_ref = refs[ix]; ix += 1
        fb_ref = refs[ix]; ix += 1
    (hw_ref, rw_ref, cb_ref, fw_ref, fbias_ref, p0g_ref, p0s_ref, p0b_ref,
     p1_ref, p1b_ref, p2_ref, p2b_ref) = refs[ix:ix + 12]
    ix += 12
    off_ref = refs[ix]; ix += 1
    st_ref = None
    if ret_state:
        st_ref = refs[ix]; ix += 1
    states_sc = refs[ix]

    m = nb * p
    feat = feat_ref[...].reshape(m, 64)
    if do_fuse:
        ct = ct_ref[...].reshape(nb, 64)
        ctv = jnp.dot(ct, wb_ref[...], preferred_element_type=jnp.float32)
        ctb = jnp.broadcast_to(ctv[:, None, :], (nb, p, 64)).reshape(m, 64)
        feat = (jnp.dot(feat, wa_ref[...], preferred_element_type=jnp.float32)
                + ctb + fb_ref[...])
    cv = cv_ref[...].reshape(m, 2)
    x = jnp.concatenate(
        [feat, cv, jnp.zeros((m, _CIN - 66), jnp.float32)], axis=1)

    xb = x.astype(jnp.bfloat16)
    for j, d in enumerate(_ALL_DILS):
        xb3 = xb.reshape(nb, p, _CIN)
        taps = jnp.concatenate(
            [_roll_rows(xb3, (-(k - _NADJ) * d) % p, p) for k in range(_K)],
            axis=2).reshape(m, _K * _CIN)
        wmat = hw_ref[...] if j == 0 else rw_ref[j - 1]
        z = jnp.dot(taps, wmat,
                    preferred_element_type=jnp.float32) + cb_ref[j]
        y = jnp.maximum(z, 0.0)
        x = y if j == 0 else y + x
        xb = x.astype(jnp.bfloat16)
        states_sc[:, j * _STATE:(j + 1) * _STATE] = xb
        if ret_state:
            lo = _FUSION + j * _STATE
            x3 = x.reshape(nb, p, _STATE)
            for b in range(nb):
                st_ref[b, lo:lo + _STATE, :] = jnp.transpose(x3[b])

    states = states_sc[...]
    fus = (jnp.dot(states, fw_ref[...], preferred_element_type=jnp.float32)
           + fbias_ref[...])
    g = jnp.max(fus.reshape(nb, p, _FUSION), axis=1)
    gg = jnp.dot(g.astype(jnp.bfloat16), p0g_ref[...],
                 preferred_element_type=jnp.float32)
    ggb = jnp.broadcast_to(gg[:, None, :], (nb, p, _FUSION)).reshape(m, _FUSION)
    h0 = jnp.maximum(
        jnp.dot(states, p0s_ref[...], preferred_element_type=jnp.float32)
        + ggb + p0b_ref[...], 0.0)
    h1 = jnp.maximum(
        jnp.dot(h0.astype(jnp.bfloat16), p1_ref[...],
                preferred_element_type=jnp.float32) + p1b_ref[...], 0.0)
    h2 = (jnp.dot(h1.astype(jnp.bfloat16), p2_ref[...],
                  preferred_element_type=jnp.float32) + p2b_ref[...])
    off_ref[...] = h2.reshape(nb, p, 2)
    if ret_state:
        for b in range(nb):
            st_ref[b, 0:_FUSION, :] = jnp.broadcast_to(
                jnp.transpose(g[b:b + 1, :]), (_FUSION, p))


def _snake_run(sp, feat_t, cvals, ct=None, fuse_w=None, ret_state=False, nb=8):
    n, p, _ = feat_t.shape
    do_fuse = ct is not None
    body = functools.partial(_snake_body, nb=nb, p=p, do_fuse=do_fuse,
                             ret_state=ret_state)
    inputs = [feat_t, cvals]
    in_specs = [pl.BlockSpec((nb, p, 64), lambda i: (i, 0, 0)),
                pl.BlockSpec((nb, p, 2), lambda i: (i, 0, 0))]
    if do_fuse:
        wa_t, wb_t, fb_t = fuse_w
        inputs += [ct, wa_t, wb_t, fb_t]
        in_specs += [pl.BlockSpec((nb, 1, 64), lambda i: (i, 0, 0)),
                     _full_spec(wa_t), _full_spec(wb_t), _full_spec(fb_t)]
    winputs = [sp['head'], sp['res'], sp['cb'], sp['fw'], sp['fb'],
               sp['p0g'], sp['p0s'], sp['p0b'], sp['p1'], sp['p1b'],
               sp['p2'], sp['p2b']]
    inputs += winputs
    in_specs += [_full_spec(a) for a in winputs]

    out_shape = [jax.ShapeDtypeStruct((n, p, 2), jnp.float32)]
    out_specs = [pl.BlockSpec((nb, p, 2), lambda i: (i, 0, 0))]
    if ret_state:
        out_shape.append(
            jax.ShapeDtypeStruct((n, _FUSION + _CAT, p), jnp.float32))
        out_specs.append(
            pl.BlockSpec((nb, _FUSION + _CAT, p), lambda i: (i, 0, 0)))

    outs = pl.pallas_call(
        body,
        out_shape=out_shape,
        grid=(n // nb,),
        in_specs=in_specs,
        out_specs=out_specs,
        scratch_shapes=[
            pltpu.VMEM((nb * p, _CAT), jnp.bfloat16),
        ],
        compiler_params=pltpu.CompilerParams(
            dimension_semantics=("parallel",),
            vmem_limit_bytes=64 * 1024 * 1024),
    )(*inputs)
    if ret_state:
        return outs[0], outs[1]
    return outs[0]


def _prep_snake(hw, rw, cb, fw, fb, p0g, p0s, p0b, p1w, p1b, p2w, p2b):
    head = jnp.zeros((_K, _CIN, _STATE), jnp.bfloat16)
    head = head.at[:, :66, :].set(jnp.swapaxes(hw, 1, 2))
    return {
        'head': head.reshape(_K * _CIN, _STATE),
        'res': jnp.swapaxes(rw, 1, 2),
        'cb': cb.reshape(8, 1, _STATE),
        'fw': jnp.swapaxes(fw, 0, 1),
        'fb': fb.reshape(1, _FUSION),
        'p0g': jnp.swapaxes(p0g, 0, 1),
        'p0s': jnp.swapaxes(p0s, 0, 1),
        'p0b': p0b.reshape(1, _FUSION),
        'p1': jnp.swapaxes(p1w, 0, 1),
        'p1b': p1b.reshape(1, 64),
        'p2': jnp.swapaxes(p2w, 0, 1),
        'p2b': p2b.reshape(1, 2),
    }


def _feat_t(cnn_pairs, img_poly, ind, h, w):
    x = img_poly[..., 0] - 0.5
    y = img_poly[..., 1] - 0.5
    x0 = jnp.floor(x)
    y0 = jnp.floor(y)
    wx1, wx0 = x - x0, 1.0 - (x - x0)
    wy1, wy0 = y - y0, 1.0 - (y - y0)
    vx0 = ((x0 >= 0) & (x0 <= w - 1)).astype(jnp.float32)
    vx1 = ((x0 >= -1) & (x0 <= w - 2)).astype(jnp.float32)
    vy0 = ((y0 >= 0) & (y0 <= h - 1)).astype(jnp.float32)
    vy1 = ((y0 >= -1) & (y0 <= h - 2)).astype(jnp.float32)
    mid = ((x0 >= 0) & (x0 <= w - 2))[..., None]
    x0c = jnp.clip(x0, 0, w - 2).astype(jnp.int32)
    y0c = jnp.clip(y0, 0, h - 1).astype(jnp.int32)
    y1c = jnp.clip(y0 + 1.0, 0, h - 1).astype(jnp.int32)
    base = ind.astype(jnp.int32)[:, None] * (h * w)
    cw0 = (wx0 * vx0)[..., None]
    cw1 = (wx1 * vx1)[..., None]

    def rowmix(yc):
        pair = cnn_pairs[base + yc * w + x0c]
        a, b = pair[..., :64], pair[..., 64:]
        px0 = jnp.where(mid, a, b)
        px1 = jnp.where(mid, b, a)
        return px0 * cw0 + px1 * cw1

    return (rowmix(y0c) * (wy0 * vy0)[..., None]
            + rowmix(y1c) * (wy1 * vy1)[..., None])


def _can_poly(img_poly):
    x_min = jnp.min(img_poly[..., 0], axis=-1, keepdims=True)
    y_min = jnp.min(img_poly[..., 1], axis=-1, keepdims=True)
    return jnp.stack([img_poly[..., 0] - x_min,
                      img_poly[..., 1] - y_min], axis=-1)


def _upsample(poly, p_num):
    n, p, _ = poly.shape
    t = jnp.arange(p_num, dtype=jnp.float32) * (p / p_num)
    i0 = jnp.floor(t).astype(jnp.int32) % p
    i1 = (i0 + 1) % p
    frac = (t - jnp.floor(t))[None, :, None]
    return poly[:, i0] * (1.0 - frac) + poly[:, i1] * frac


def kernel(cnn_feature, i_it_4py, c_it_4py, ind, shift_init, fuse_wa, fuse_wb, fuse_b, g0_head_w, g0_res_w, g0_conv_b, g0_fusion_w, g0_fusion_b, g0_pred0_gw, g0_pred0_sw, g0_pred0_b, g0_pred1_w, g0_pred1_b, g0_pred2_w, g0_pred2_b, g1_head_w, g1_res_w, g1_conv_b, g1_fusion_w, g1_fusion_b, g1_pred0_gw, g1_pred0_sw, g1_pred0_b, g1_pred1_w, g1_pred1_b, g1_pred2_w, g1_pred2_b, g2_head_w, g2_res_w, g2_conv_b, g2_fusion_w, g2_fusion_b, g2_pred0_gw, g2_pred0_sw, g2_pred0_b, g2_pred1_w, g2_pred1_b, g2_pred2_w, g2_pred2_b, ge_head_w, ge_res_w, ge_conv_b, ge_fusion_w, ge_fusion_b, ge_pred0_gw, ge_pred0_sw, ge_pred0_b, ge_pred1_w, ge_pred1_b, ge_pred2_w, ge_pred2_b, ge0_head_w, ge0_res_w, ge0_conv_b, ge0_fusion_w, ge0_fusion_b, ge0_pred0_gw, ge0_pred0_sw, ge0_pred0_b, ge0_pred1_w, ge0_pred1_b, ge0_pred2_w, ge0_pred2_b):
    h, w = cnn_feature.shape[2], cnn_feature.shape[3]
    cnn_rows = jnp.transpose(cnn_feature, (0, 2, 3, 1)).reshape(-1, 64)
    cnn_pairs = jnp.concatenate(
        [cnn_rows, jnp.concatenate([cnn_rows[1:], cnn_rows[:1]], axis=0)],
        axis=1)
    fuse = (jnp.swapaxes(fuse_wa, 0, 1), jnp.swapaxes(fuse_wb, 0, 1),
            fuse_b.reshape(1, 64))
    sps = [
        _prep_snake(g0_head_w, g0_res_w, g0_conv_b, g0_fusion_w, g0_fusion_b,
                    g0_pred0_gw, g0_pred0_sw, g0_pred0_b, g0_pred1_w,
                    g0_pred1_b, g0_pred2_w, g0_pred2_b),
        _prep_snake(g1_head_w, g1_res_w, g1_conv_b, g1_fusion_w, g1_fusion_b,
                    g1_pred0_gw, g1_pred0_sw, g1_pred0_b, g1_pred1_w,
                    g1_pred1_b, g1_pred2_w, g1_pred2_b),
        _prep_snake(g2_head_w, g2_res_w, g2_conv_b, g2_fusion_w, g2_fusion_b,
                    g2_pred0_gw, g2_pred0_sw, g2_pred0_b, g2_pred1_w,
                    g2_pred1_b, g2_pred2_w, g2_pred2_b),
        _prep_snake(ge_head_w, ge_res_w, ge_conv_b, ge_fusion_w, ge_fusion_b,
                    ge_pred0_gw, ge_pred0_sw, ge_pred0_b, ge_pred1_w,
                    ge_pred1_b, ge_pred2_w, ge_pred2_b),
        _prep_snake(ge0_head_w, ge0_res_w, ge0_conv_b, ge0_fusion_w,
                    ge0_fusion_b, ge0_pred0_gw, ge0_pred0_sw, ge0_pred0_b,
                    ge0_pred1_w, ge0_pred1_b, ge0_pred2_w, ge0_pred2_b),
    ]

    ipoly = i_it_4py
    cpoly = c_it_4py
    for it in range(3):
        feat_t = _feat_t(cnn_pairs, ipoly, ind, h, w)
        center = (jnp.min(ipoly, axis=1) + jnp.max(ipoly, axis=1)) * 0.5
        ct_t = _feat_t(cnn_pairs, center[:, None], ind, h, w)
        off = _snake_run(sps[it], feat_t, cpoly, ct=ct_t, fuse_w=fuse, nb=32)
        ipoly = ipoly + off
        cpoly = _can_poly(ipoly)
    ex_pred = ipoly

    i_py = _upsample(ex_pred, _POLY)
    c_py = _can_poly(i_py)
    feat_t = _feat_t(cnn_pairs, i_py, ind, h, w)
    off = _snake_run(sps[3], feat_t, c_py * _RO, nb=16)
    py0 = i_py * _RO + off

    cur = py0 / _RO
    c_cur = _can_poly(cur)
    feat_t = _feat_t(cnn_pairs, cur, ind, h, w)
    off, st_t = _snake_run(sps[4], feat_t, c_cur * _RO, ret_state=True, nb=16)
    py1 = cur * _RO + off
    state = st_t

    return {
        'ex_pred': ex_pred,
        'py_pred': [py0, py1],
        'state': state,
        'py': [py0 / _RO, py1 / _RO],
    }


# speedup vs baseline: 1.1290x; 1.1226x over previous
import functools

import jax
import jax.numpy as jnp
from jax.experimental import pallas as pl
from jax.experimental.pallas import tpu as pltpu

_RO = 4.0
_POLY = 128
_NADJ = 4
_K = 9
_RES_DILS = (1, 1, 1, 2, 2, 4, 4)
_ALL_DILS = (1,) + _RES_DILS
_CIN = 128
_STATE = 128
_FUSION = 256
_CAT = 8 * _STATE


def _full_spec(arr):
    zeros = (0,) * arr.ndim
    return pl.BlockSpec(arr.shape, lambda i, _z=zeros: _z)


def _roll_rows(a, s, p):
    if s == 0:
        return a
    return jnp.concatenate([a[:, p - s:, :], a[:, :p - s, :]], axis=1)


def _snake_body(*refs, nb, p, do_fuse, ret_state):
    ix = 0
    feat_ref = refs[ix]; ix += 1
    cv_ref = refs[ix]; ix += 1
    if do_fuse:
        ct_ref = refs[ix]; ix += 1
        wa_ref = refs[ix]; ix += 1
        wb_ref = refs[ix]; ix += 1
        fb_ref = refs[ix]; ix += 1
    (hw_ref, rw_ref, cb_ref, fw_ref, fbias_ref, p0g_ref, p0s_ref, p0b_ref,
     p1_ref, p1b_ref, p2_ref, p2b_ref) = refs[ix:ix + 12]
    ix += 12
    off_ref = refs[ix]; ix += 1
    st_ref = None
    if ret_state:
        st_ref = refs[ix]; ix += 1
    states_sc = refs[ix]

    m = nb * p
    feat = feat_ref[...].reshape(m, 64)
    if do_fuse:
        ct = ct_ref[...].reshape(nb, 64)
        ctv = jnp.dot(ct, wb_ref[...], preferred_element_type=jnp.float32)
        ctb = jnp.broadcast_to(ctv[:, None, :], (nb, p, 64)).reshape(m, 64)
        feat = (jnp.dot(feat, wa_ref[...], preferred_element_type=jnp.float32)
                + ctb + fb_ref[...])
    cv = cv_ref[...].reshape(m, 2)
    x = jnp.concatenate(
        [feat, cv, jnp.zeros((m, _CIN - 66), jnp.float32)], axis=1)

    nh = nb // 2 if nb % 2 == 0 else nb
    xb = x.astype(jnp.bfloat16)
    for j, d in enumerate(_ALL_DILS):
        xb3 = xb.reshape(nb, p, _CIN)
        wmat = hw_ref[...] if j == 0 else rw_ref[j - 1]
        zs = []
        for lo in range(0, nb, nh):
            sub = xb3[lo:lo + nh]
            taps = jnp.concatenate(
                [_roll_rows(sub, (-(k - _NADJ) * d) % p, p)
                 for k in range(_K)],
                axis=2).reshape(nh * p, _K * _CIN)
            zs.append(jnp.dot(taps, wmat,
                              preferred_element_type=jnp.float32))
        z = jnp.concatenate(zs, axis=0) + cb_ref[j]
        y = jnp.maximum(z, 0.0)
        x = y if j == 0 else y + x
        xb = x.astype(jnp.bfloat16)
        states_sc[:, j * _STATE:(j + 1) * _STATE] = xb
        if ret_state:
            lo = _FUSION + j * _STATE
            x3 = x.reshape(nb, p, _STATE)
            for b in range(nb):
                st_ref[b, lo:lo + _STATE, :] = jnp.transpose(x3[b])

    states = states_sc[...]
    fus = (jnp.dot(states, fw_ref[...], preferred_element_type=jnp.float32)
           + fbias_ref[...])
    g = jnp.max(fus.reshape(nb, p, _FUSION), axis=1)
    gg = jnp.dot(g.astype(jnp.bfloat16), p0g_ref[...],
                 preferred_element_type=jnp.float32)
    ggb = jnp.broadcast_to(gg[:, None, :], (nb, p, _FUSION)).reshape(m, _FUSION)
    h0 = jnp.maximum(
        jnp.dot(states, p0s_ref[...], preferred_element_type=jnp.float32)
        + ggb + p0b_ref[...], 0.0)
    h1 = jnp.maximum(
        jnp.dot(h0.astype(jnp.bfloat16), p1_ref[...],
                preferred_element_type=jnp.float32) + p1b_ref[...], 0.0)
    h2 = (jnp.dot(h1.astype(jnp.bfloat16), p2_ref[...],
                  preferred_element_type=jnp.float32) + p2b_ref[...])
    off_ref[...] = h2.reshape(nb, p, 2)
    if ret_state:
        for b in range(nb):
            st_ref[b, 0:_FUSION, :] = jnp.broadcast_to(
                jnp.transpose(g[b:b + 1, :]), (_FUSION, p))


def _snake_run(sp, feat_t, cvals, ct=None, fuse_w=None, ret_state=False, nb=8):
    n, p, _ = feat_t.shape
    do_fuse = ct is not None
    body = functools.partial(_snake_body, nb=nb, p=p, do_fuse=do_fuse,
                             ret_state=ret_state)
    inputs = [feat_t, cvals]
    in_specs = [pl.BlockSpec((nb, p, 64), lambda i: (i, 0, 0)),
                pl.BlockSpec((nb, p, 2), lambda i: (i, 0, 0))]
    if do_fuse:
        wa_t, wb_t, fb_t = fuse_w
        inputs += [ct, wa_t, wb_t, fb_t]
        in_specs += [pl.BlockSpec((nb, 1, 64), lambda i: (i, 0, 0)),
                     _full_spec(wa_t), _full_spec(wb_t), _full_spec(fb_t)]
    winputs = [sp['head'], sp['res'], sp['cb'], sp['fw'], sp['fb'],
               sp['p0g'], sp['p0s'], sp['p0b'], sp['p1'], sp['p1b'],
               sp['p2'], sp['p2b']]
    inputs += winputs
    in_specs += [_full_spec(a) for a in winputs]

    out_shape = [jax.ShapeDtypeStruct((n, p, 2), jnp.float32)]
    out_specs = [pl.BlockSpec((nb, p, 2), lambda i: (i, 0, 0))]
    if ret_state:
        out_shape.append(
            jax.ShapeDtypeStruct((n, _FUSION + _CAT, p), jnp.float32))
        out_specs.append(
            pl.BlockSpec((nb, _FUSION + _CAT, p), lambda i: (i, 0, 0)))

    outs = pl.pallas_call(
        body,
        out_shape=out_shape,
        grid=(n // nb,),
        in_specs=in_specs,
        out_specs=out_specs,
        scratch_shapes=[
            pltpu.VMEM((nb * p, _CAT), jnp.bfloat16),
        ],
        compiler_params=pltpu.CompilerParams(
            dimension_semantics=("parallel",),
            vmem_limit_bytes=64 * 1024 * 1024),
    )(*inputs)
    if ret_state:
        return outs[0], outs[1]
    return outs[0]


def _prep_snake(hw, rw, cb, fw, fb, p0g, p0s, p0b, p1w, p1b, p2w, p2b):
    head = jnp.zeros((_K, _CIN, _STATE), jnp.bfloat16)
    head = head.at[:, :66, :].set(jnp.swapaxes(hw, 1, 2))
    return {
        'head': head.reshape(_K * _CIN, _STATE),
        'res': jnp.swapaxes(rw, 1, 2),
        'cb': cb.reshape(8, 1, _STATE),
        'fw': jnp.swapaxes(fw, 0, 1),
        'fb': fb.reshape(1, _FUSION),
        'p0g': jnp.swapaxes(p0g, 0, 1),
        'p0s': jnp.swapaxes(p0s, 0, 1),
        'p0b': p0b.reshape(1, _FUSION),
        'p1': jnp.swapaxes(p1w, 0, 1),
        'p1b': p1b.reshape(1, 64),
        'p2': jnp.swapaxes(p2w, 0, 1),
        'p2b': p2b.reshape(1, 2),
    }


def _feat_t(cnn_pairs, img_poly, ind, h, w):
    x = img_poly[..., 0] - 0.5
    y = img_poly[..., 1] - 0.5
    x0 = jnp.floor(x)
    y0 = jnp.floor(y)
    wx1, wx0 = x - x0, 1.0 - (x - x0)
    wy1, wy0 = y - y0, 1.0 - (y - y0)
    vx0 = ((x0 >= 0) & (x0 <= w - 1)).astype(jnp.float32)
    vx1 = ((x0 >= -1) & (x0 <= w - 2)).astype(jnp.float32)
    vy0 = ((y0 >= 0) & (y0 <= h - 1)).astype(jnp.float32)
    vy1 = ((y0 >= -1) & (y0 <= h - 2)).astype(jnp.float32)
    mid = ((x0 >= 0) & (x0 <= w - 2))[..., None]
    x0c = jnp.clip(x0, 0, w - 2).astype(jnp.int32)
    y0c = jnp.clip(y0, 0, h - 1).astype(jnp.int32)
    y1c = jnp.clip(y0 + 1.0, 0, h - 1).astype(jnp.int32)
    base = ind.astype(jnp.int32)[:, None] * (h * w)
    cw0 = (wx0 * vx0)[..., None]
    cw1 = (wx1 * vx1)[..., None]

    def rowmix(yc):
        pair = cnn_pairs[base + yc * w + x0c]
        a, b = pair[..., :64], pair[..., 64:]
        px0 = jnp.where(mid, a, b)
        px1 = jnp.where(mid, b, a)
        return px0 * cw0 + px1 * cw1

    return (rowmix(y0c) * (wy0 * vy0)[..., None]
            + rowmix(y1c) * (wy1 * vy1)[..., None])


def _can_poly(img_poly):
    x_min = jnp.min(img_poly[..., 0], axis=-1, keepdims=True)
    y_min = jnp.min(img_poly[..., 1], axis=-1, keepdims=True)
    return jnp.stack([img_poly[..., 0] - x_min,
                      img_poly[..., 1] - y_min], axis=-1)


def _upsample(poly, p_num):
    n, p, _ = poly.shape
    t = jnp.arange(p_num, dtype=jnp.float32) * (p / p_num)
    i0 = jnp.floor(t).astype(jnp.int32) % p
    i1 = (i0 + 1) % p
    frac = (t - jnp.floor(t))[None, :, None]
    return poly[:, i0] * (1.0 - frac) + poly[:, i1] * frac


def kernel(cnn_feature, i_it_4py, c_it_4py, ind, shift_init, fuse_wa, fuse_wb, fuse_b, g0_head_w, g0_res_w, g0_conv_b, g0_fusion_w, g0_fusion_b, g0_pred0_gw, g0_pred0_sw, g0_pred0_b, g0_pred1_w, g0_pred1_b, g0_pred2_w, g0_pred2_b, g1_head_w, g1_res_w, g1_conv_b, g1_fusion_w, g1_fusion_b, g1_pred0_gw, g1_pred0_sw, g1_pred0_b, g1_pred1_w, g1_pred1_b, g1_pred2_w, g1_pred2_b, g2_head_w, g2_res_w, g2_conv_b, g2_fusion_w, g2_fusion_b, g2_pred0_gw, g2_pred0_sw, g2_pred0_b, g2_pred1_w, g2_pred1_b, g2_pred2_w, g2_pred2_b, ge_head_w, ge_res_w, ge_conv_b, ge_fusion_w, ge_fusion_b, ge_pred0_gw, ge_pred0_sw, ge_pred0_b, ge_pred1_w, ge_pred1_b, ge_pred2_w, ge_pred2_b, ge0_head_w, ge0_res_w, ge0_conv_b, ge0_fusion_w, ge0_fusion_b, ge0_pred0_gw, ge0_pred0_sw, ge0_pred0_b, ge0_pred1_w, ge0_pred1_b, ge0_pred2_w, ge0_pred2_b):
    h, w = cnn_feature.shape[2], cnn_feature.shape[3]
    cnn_rows = jnp.transpose(cnn_feature, (0, 2, 3, 1)).reshape(-1, 64)
    cnn_pairs = jnp.concatenate(
        [cnn_rows, jnp.concatenate([cnn_rows[1:], cnn_rows[:1]], axis=0)],
        axis=1)
    fuse = (jnp.swapaxes(fuse_wa, 0, 1), jnp.swapaxes(fuse_wb, 0, 1),
            fuse_b.reshape(1, 64))
    sps = [
        _prep_snake(g0_head_w, g0_res_w, g0_conv_b, g0_fusion_w, g0_fusion_b,
                    g0_pred0_gw, g0_pred0_sw, g0_pred0_b, g0_pred1_w,
                    g0_pred1_b, g0_pred2_w, g0_pred2_b),
        _prep_snake(g1_head_w, g1_res_w, g1_conv_b, g1_fusion_w, g1_fusion_b,
                    g1_pred0_gw, g1_pred0_sw, g1_pred0_b, g1_pred1_w,
                    g1_pred1_b, g1_pred2_w, g1_pred2_b),
        _prep_snake(g2_head_w, g2_res_w, g2_conv_b, g2_fusion_w, g2_fusion_b,
                    g2_pred0_gw, g2_pred0_sw, g2_pred0_b, g2_pred1_w,
                    g2_pred1_b, g2_pred2_w, g2_pred2_b),
        _prep_snake(ge_head_w, ge_res_w, ge_conv_b, ge_fusion_w, ge_fusion_b,
                    ge_pred0_gw, ge_pred0_sw, ge_pred0_b, ge_pred1_w,
                    ge_pred1_b, ge_pred2_w, ge_pred2_b),
        _prep_snake(ge0_head_w, ge0_res_w, ge0_conv_b, ge0_fusion_w,
                    ge0_fusion_b, ge0_pred0_gw, ge0_pred0_sw, ge0_pred0_b,
                    ge0_pred1_w, ge0_pred1_b, ge0_pred2_w, ge0_pred2_b),
    ]

    ipoly = i_it_4py
    cpoly = c_it_4py
    for it in range(3):
        feat_t = _feat_t(cnn_pairs, ipoly, ind, h, w)
        center = (jnp.min(ipoly, axis=1) + jnp.max(ipoly, axis=1)) * 0.5
        ct_t = _feat_t(cnn_pairs, center[:, None], ind, h, w)
        off = _snake_run(sps[it], feat_t, cpoly, ct=ct_t, fuse_w=fuse, nb=32)
        ipoly = ipoly + off
        cpoly = _can_poly(ipoly)
    ex_pred = ipoly

    i_py = _upsample(ex_pred, _POLY)
    c_py = _can_poly(i_py)
    feat_t = _feat_t(cnn_pairs, i_py, ind, h, w)
    off = _snake_run(sps[3], feat_t, c_py * _RO, nb=16)
    py0 = i_py * _RO + off

    cur = py0 / _RO
    c_cur = _can_poly(cur)
    feat_t = _feat_t(cnn_pairs, cur, ind, h, w)
    off, st_t = _snake_run(sps[4], feat_t, c_cur * _RO, ret_state=True, nb=16)
    py1 = cur * _RO + off
    state = st_t

    return {
        'ex_pred': ex_pred,
        'py_pred': [py0, py1],
        'state': state,
        'py': [py0 / _RO, py1 / _RO],
    }


# speedup vs baseline: 1.1363x; 1.0065x over previous
import functools

import jax
import jax.numpy as jnp
from jax.experimental import pallas as pl
from jax.experimental.pallas import tpu as pltpu

_RO = 4.0
_POLY = 128
_NADJ = 4
_K = 9
_RES_DILS = (1, 1, 1, 2, 2, 4, 4)
_ALL_DILS = (1,) + _RES_DILS
_CIN = 128
_STATE = 128
_FUSION = 256
_CAT = 8 * _STATE


def _full_spec(arr):
    zeros = (0,) * arr.ndim
    return pl.BlockSpec(arr.shape, lambda i, _z=zeros: _z)


def _roll_rows(a, s, p):
    if s == 0:
        return a
    return jnp.concatenate([a[:, p - s:, :], a[:, :p - s, :]], axis=1)


def _snake_body(*refs, nb, p, do_fuse, ret_state):
    ix = 0
    feat_ref = refs[ix]; ix += 1
    cv_ref = refs[ix]; ix += 1
    if do_fuse:
        ct_ref = refs[ix]; ix += 1
        wa_ref = refs[ix]; ix += 1
        wb_ref = refs[ix]; ix += 1
        fb_ref = refs[ix]; ix += 1
    (hw_ref, rw_ref, cb_ref, fw_ref, fbias_ref, p0g_ref, p0s_ref, p0b_ref,
     p1_ref, p1b_ref, p2_ref, p2b_ref) = refs[ix:ix + 12]
    ix += 12
    off_ref = refs[ix]; ix += 1
    st_ref = None
    if ret_state:
        st_ref = refs[ix]; ix += 1
    states_sc = refs[ix]

    m = nb * p
    feat = feat_ref[...].reshape(m, 64)
    if do_fuse:
        ct = ct_ref[...].reshape(nb, 64)
        ctv = jnp.dot(ct, wb_ref[...], preferred_element_type=jnp.float32)
        ctb = jnp.broadcast_to(ctv[:, None, :], (nb, p, 64)).reshape(m, 64)
        feat = (jnp.dot(feat, wa_ref[...], preferred_element_type=jnp.float32)
                + ctb + fb_ref[...])
    cv = cv_ref[...].reshape(m, 2)
    x = jnp.concatenate(
        [feat, cv, jnp.zeros((m, _CIN - 66), jnp.float32)], axis=1)

    nh = nb // 4 if nb % 4 == 0 else nb
    xb = x.astype(jnp.bfloat16)
    for j, d in enumerate(_ALL_DILS):
        xb3 = xb.reshape(nb, p, _CIN)
        wmat = hw_ref[...] if j == 0 else rw_ref[j - 1]
        zs = []
        for lo in range(0, nb, nh):
            sub = xb3[lo:lo + nh]
            taps = jnp.concatenate(
                [_roll_rows(sub, (-(k - _NADJ) * d) % p, p)
                 for k in range(_K)],
                axis=2).reshape(nh * p, _K * _CIN)
            zs.append(jnp.dot(taps, wmat,
                              preferred_element_type=jnp.float32))
        z = jnp.concatenate(zs, axis=0) + cb_ref[j]
        y = jnp.maximum(z, 0.0)
        x = y if j == 0 else y + x
        xb = x.astype(jnp.bfloat16)
        states_sc[:, j * _STATE:(j + 1) * _STATE] = xb
        if ret_state:
            lo = _FUSION + j * _STATE
            x3 = x.reshape(nb, p, _STATE)
            for b in range(nb):
                st_ref[b, lo:lo + _STATE, :] = jnp.transpose(x3[b])

    states = states_sc[...]
    fus = (jnp.dot(states, fw_ref[...], preferred_element_type=jnp.float32)
           + fbias_ref[...])
    g = jnp.max(fus.reshape(nb, p, _FUSION), axis=1)
    gg = jnp.dot(g.astype(jnp.bfloat16), p0g_ref[...],
                 preferred_element_type=jnp.float32)
    ggb = jnp.broadcast_to(gg[:, None, :], (nb, p, _FUSION)).reshape(m, _FUSION)
    h0 = jnp.maximum(
        jnp.dot(states, p0s_ref[...], preferred_element_type=jnp.float32)
        + ggb + p0b_ref[...], 0.0)
    h1 = jnp.maximum(
        jnp.dot(h0.astype(jnp.bfloat16), p1_ref[...],
                preferred_element_type=jnp.float32) + p1b_ref[...], 0.0)
    h2 = (jnp.dot(h1.astype(jnp.bfloat16), p2_ref[...],
                  preferred_element_type=jnp.float32) + p2b_ref[...])
    off_ref[...] = h2.reshape(nb, p, 2)
    if ret_state:
        for b in range(nb):
            st_ref[b, 0:_FUSION, :] = jnp.broadcast_to(
                jnp.transpose(g[b:b + 1, :]), (_FUSION, p))


def _snake_run(sp, feat_t, cvals, ct=None, fuse_w=None, ret_state=False, nb=8):
    n, p, _ = feat_t.shape
    do_fuse = ct is not None
    body = functools.partial(_snake_body, nb=nb, p=p, do_fuse=do_fuse,
                             ret_state=ret_state)
    inputs = [feat_t, cvals]
    in_specs = [pl.BlockSpec((nb, p, 64), lambda i: (i, 0, 0)),
                pl.BlockSpec((nb, p, 2), lambda i: (i, 0, 0))]
    if do_fuse:
        wa_t, wb_t, fb_t = fuse_w
        inputs += [ct, wa_t, wb_t, fb_t]
        in_specs += [pl.BlockSpec((nb, 1, 64), lambda i: (i, 0, 0)),
                     _full_spec(wa_t), _full_spec(wb_t), _full_spec(fb_t)]
    winputs = [sp['head'], sp['res'], sp['cb'], sp['fw'], sp['fb'],
               sp['p0g'], sp['p0s'], sp['p0b'], sp['p1'], sp['p1b'],
               sp['p2'], sp['p2b']]
    inputs += winputs
    in_specs += [_full_spec(a) for a in winputs]

    out_shape = [jax.ShapeDtypeStruct((n, p, 2), jnp.float32)]
    out_specs = [pl.BlockSpec((nb, p, 2), lambda i: (i, 0, 0))]
    if ret_state:
        out_shape.append(
            jax.ShapeDtypeStruct((n, _FUSION + _CAT, p), jnp.float32))
        out_specs.append(
            pl.BlockSpec((nb, _FUSION + _CAT, p), lambda i: (i, 0, 0)))

    outs = pl.pallas_call(
        body,
        out_shape=out_shape,
        grid=(n // nb,),
        in_specs=in_specs,
        out_specs=out_specs,
        scratch_shapes=[
            pltpu.VMEM((nb * p, _CAT), jnp.bfloat16),
        ],
        compiler_params=pltpu.CompilerParams(
            dimension_semantics=("parallel",),
            vmem_limit_bytes=64 * 1024 * 1024),
    )(*inputs)
    if ret_state:
        return outs[0], outs[1]
    return outs[0]


def _prep_snake(hw, rw, cb, fw, fb, p0g, p0s, p0b, p1w, p1b, p2w, p2b):
    head = jnp.zeros((_K, _CIN, _STATE), jnp.bfloat16)
    head = head.at[:, :66, :].set(jnp.swapaxes(hw, 1, 2))
    return {
        'head': head.reshape(_K * _CIN, _STATE),
        'res': jnp.swapaxes(rw, 1, 2),
        'cb': cb.reshape(8, 1, _STATE),
        'fw': jnp.swapaxes(fw, 0, 1),
        'fb': fb.reshape(1, _FUSION),
        'p0g': jnp.swapaxes(p0g, 0, 1),
        'p0s': jnp.swapaxes(p0s, 0, 1),
        'p0b': p0b.reshape(1, _FUSION),
        'p1': jnp.swapaxes(p1w, 0, 1),
        'p1b': p1b.reshape(1, 64),
        'p2': jnp.swapaxes(p2w, 0, 1),
        'p2b': p2b.reshape(1, 2),
    }


def _feat_t(cnn_pairs, img_poly, ind, h, w):
    x = img_poly[..., 0] - 0.5
    y = img_poly[..., 1] - 0.5
    x0 = jnp.floor(x)
    y0 = jnp.floor(y)
    wx1, wx0 = x - x0, 1.0 - (x - x0)
    wy1, wy0 = y - y0, 1.0 - (y - y0)
    vx0 = ((x0 >= 0) & (x0 <= w - 1)).astype(jnp.float32)
    vx1 = ((x0 >= -1) & (x0 <= w - 2)).astype(jnp.float32)
    vy0 = ((y0 >= 0) & (y0 <= h - 1)).astype(jnp.float32)
    vy1 = ((y0 >= -1) & (y0 <= h - 2)).astype(jnp.float32)
    mid = ((x0 >= 0) & (x0 <= w - 2))[..., None]
    x0c = jnp.clip(x0, 0, w - 2).astype(jnp.int32)
    y0c = jnp.clip(y0, 0, h - 1).astype(jnp.int32)
    y1c = jnp.clip(y0 + 1.0, 0, h - 1).astype(jnp.int32)
    base = ind.astype(jnp.int32)[:, None] * (h * w)
    cw0 = (wx0 * vx0)[..., None]
    cw1 = (wx1 * vx1)[..., None]

    def rowmix(yc):
        pair = cnn_pairs[base + yc * w + x0c]
        a, b = pair[..., :64], pair[..., 64:]
        px0 = jnp.where(mid, a, b)
        px1 = jnp.where(mid, b, a)
        return px0 * cw0 + px1 * cw1

    return (rowmix(y0c) * (wy0 * vy0)[..., None]
            + rowmix(y1c) * (wy1 * vy1)[..., None])


def _can_poly(img_poly):
    x_min = jnp.min(img_poly[..., 0], axis=-1, keepdims=True)
    y_min = jnp.min(img_poly[..., 1], axis=-1, keepdims=True)
    return jnp.stack([img_poly[..., 0] - x_min,
                      img_poly[..., 1] - y_min], axis=-1)


def _upsample(poly, p_num):
    n, p, _ = poly.shape
    t = jnp.arange(p_num, dtype=jnp.float32) * (p / p_num)
    i0 = jnp.floor(t).astype(jnp.int32) % p
    i1 = (i0 + 1) % p
    frac = (t - jnp.floor(t))[None, :, None]
    return poly[:, i0] * (1.0 - frac) + poly[:, i1] * frac


def kernel(cnn_feature, i_it_4py, c_it_4py, ind, shift_init, fuse_wa, fuse_wb, fuse_b, g0_head_w, g0_res_w, g0_conv_b, g0_fusion_w, g0_fusion_b, g0_pred0_gw, g0_pred0_sw, g0_pred0_b, g0_pred1_w, g0_pred1_b, g0_pred2_w, g0_pred2_b, g1_head_w, g1_res_w, g1_conv_b, g1_fusion_w, g1_fusion_b, g1_pred0_gw, g1_pred0_sw, g1_pred0_b, g1_pred1_w, g1_pred1_b, g1_pred2_w, g1_pred2_b, g2_head_w, g2_res_w, g2_conv_b, g2_fusion_w, g2_fusion_b, g2_pred0_gw, g2_pred0_sw, g2_pred0_b, g2_pred1_w, g2_pred1_b, g2_pred2_w, g2_pred2_b, ge_head_w, ge_res_w, ge_conv_b, ge_fusion_w, ge_fusion_b, ge_pred0_gw, ge_pred0_sw, ge_pred0_b, ge_pred1_w, ge_pred1_b, ge_pred2_w, ge_pred2_b, ge0_head_w, ge0_res_w, ge0_conv_b, ge0_fusion_w, ge0_fusion_b, ge0_pred0_gw, ge0_pred0_sw, ge0_pred0_b, ge0_pred1_w, ge0_pred1_b, ge0_pred2_w, ge0_pred2_b):
    h, w = cnn_feature.shape[2], cnn_feature.shape[3]
    cnn_rows = jnp.transpose(cnn_feature, (0, 2, 3, 1)).reshape(-1, 64)
    cnn_pairs = jnp.concatenate(
        [cnn_rows, jnp.concatenate([cnn_rows[1:], cnn_rows[:1]], axis=0)],
        axis=1)
    fuse = (jnp.swapaxes(fuse_wa, 0, 1), jnp.swapaxes(fuse_wb, 0, 1),
            fuse_b.reshape(1, 64))
    sps = [
        _prep_snake(g0_head_w, g0_res_w, g0_conv_b, g0_fusion_w, g0_fusion_b,
                    g0_pred0_gw, g0_pred0_sw, g0_pred0_b, g0_pred1_w,
                    g0_pred1_b, g0_pred2_w, g0_pred2_b),
        _prep_snake(g1_head_w, g1_res_w, g1_conv_b, g1_fusion_w, g1_fusion_b,
                    g1_pred0_gw, g1_pred0_sw, g1_pred0_b, g1_pred1_w,
                    g1_pred1_b, g1_pred2_w, g1_pred2_b),
        _prep_snake(g2_head_w, g2_res_w, g2_conv_b, g2_fusion_w, g2_fusion_b,
                    g2_pred0_gw, g2_pred0_sw, g2_pred0_b, g2_pred1_w,
                    g2_pred1_b, g2_pred2_w, g2_pred2_b),
        _prep_snake(ge_head_w, ge_res_w, ge_conv_b, ge_fusion_w, ge_fusion_b,
                    ge_pred0_gw, ge_pred0_sw, ge_pred0_b, ge_pred1_w,
                    ge_pred1_b, ge_pred2_w, ge_pred2_b),
        _prep_snake(ge0_head_w, ge0_res_w, ge0_conv_b, ge0_fusion_w,
                    ge0_fusion_b, ge0_pred0_gw, ge0_pred0_sw, ge0_pred0_b,
                    ge0_pred1_w, ge0_pred1_b, ge0_pred2_w, ge0_pred2_b),
    ]

    ipoly = i_it_4py
    cpoly = c_it_4py
    for it in range(3):
        feat_t = _feat_t(cnn_pairs, ipoly, ind, h, w)
        center = (jnp.min(ipoly, axis=1) + jnp.max(ipoly, axis=1)) * 0.5
        ct_t = _feat_t(cnn_pairs, center[:, None], ind, h, w)
        off = _snake_run(sps[it], feat_t, cpoly, ct=ct_t, fuse_w=fuse, nb=32)
        ipoly = ipoly + off
        cpoly = _can_poly(ipoly)
    ex_pred = ipoly

    i_py = _upsample(ex_pred, _POLY)
    c_py = _can_poly(i_py)
    feat_t = _feat_t(cnn_pairs, i_py, ind, h, w)
    off = _snake_run(sps[3], feat_t, c_py * _RO, nb=16)
    py0 = i_py * _RO + off

    cur = py0 / _RO
    c_cur = _can_poly(cur)
    feat_t = _feat_t(cnn_pairs, cur, ind, h, w)
    off, st_t = _snake_run(sps[4], feat_t, c_cur * _RO, ret_state=True, nb=16)
    py1 = cur * _RO + off
    state = st_t

    return {
        'ex_pred': ex_pred,
        'py_pred': [py0, py1],
        'state': state,
        'py': [py0 / _RO, py1 / _RO],
    }
